# baseline (speedup 1.0000x reference)
import functools

import jax
import jax.numpy as jnp
from jax import lax
from jax.experimental import pallas as pl
from jax.experimental.pallas import tpu as pltpu
from jax.experimental.pallas import tpu_sc as plsc

NC = 2
NS = 16
L = 16


def _ee_body(bpw, d, uidx_hbm, iidx_hbm, eu_hbm, ei_hbm, bu_hbm, bi_hbm,
             gm_hbm, out_hbm, uidx_v, iidx_v, urows_v, irows_v, ubias_v,
             ibias_v, gm_v, out_v, sem):
  wid = lax.axis_index("s") * NC + lax.axis_index("c")
  base = wid * bpw

  pltpu.sync_copy(uidx_hbm.at[pl.ds(base, bpw)], uidx_v)
  pltpu.sync_copy(iidx_hbm.at[pl.ds(base, bpw)], iidx_v)
  pltpu.sync_copy(gm_hbm, gm_v)

  c1 = pltpu.async_copy(eu_hbm.at[uidx_v], urows_v, sem)
  c2 = pltpu.async_copy(ei_hbm.at[iidx_v], irows_v, sem)
  c3 = pltpu.async_copy(bu_hbm.at[uidx_v], ubias_v, sem)
  c4 = pltpu.async_copy(bi_hbm.at[iidx_v], ibias_v, sem)
  c1.wait()
  c2.wait()
  c3.wait()
  c4.wait()

  gm = gm_v[...]
  lane = lax.iota(jnp.int32, L)

  def group(g, carry):
    row = lane + g * L
    acc = jnp.zeros((L,), jnp.float32)
    for dd in range(d):
      col = jnp.full((L,), dd, jnp.int32)
      uv = plsc.load_gather(urows_v, [row, col])
      iv = plsc.load_gather(irows_v, [row, col])
      df = uv - iv
      acc = acc + df * df
    accs = jnp.maximum(acc, jnp.float32(1e-30))
    yi = jnp.int32(0x5F3759DF) - lax.shift_right_logical(
        plsc.bitcast(accs, jnp.int32), 1)
    y = plsc.bitcast(yi, jnp.float32)
    for _ in range(3):
      y = y * (jnp.float32(1.5) - jnp.float32(0.5) * accs * y * y)
    dist = acc * y
    ub = ubias_v[pl.ds(g * L, L)]
    ib = ibias_v[pl.ds(g * L, L)]
    x = gm + ub + ib - dist
    out_v[pl.ds(g * L, L)] = jnp.float32(1.0) / (jnp.float32(1.0) +
                                                 jnp.exp(-x))
    return carry

  lax.fori_loop(0, bpw // L, group, 0)
  pltpu.sync_copy(out_v, out_hbm.at[pl.ds(base, bpw)])


def kernel(user_indices, item_indices, embedding_user, embedding_item,
           bias_user, bias_item, global_mean=0.0):
  b = user_indices.shape[0]
  d = embedding_user.shape[1]
  nw = NC * NS
  bpw = b // nw
  uidx = user_indices.astype(jnp.int32)
  iidx = item_indices.astype(jnp.int32)
  gm_arr = jnp.full((L,), global_mean, jnp.float32)

  mesh = plsc.VectorSubcoreMesh(core_axis_name="c", subcore_axis_name="s",
                                num_cores=NC, num_subcores=NS)
  run = pl.kernel(
      functools.partial(_ee_body, bpw, d),
      out_type=jax.ShapeDtypeStruct((b,), jnp.float32),
      mesh=mesh,
      compiler_params=pltpu.CompilerParams(needs_layout_passes=False,
                                           use_tc_tiling_on_sc=False),
      scratch_types=[
          pltpu.VMEM((bpw,), jnp.int32),
          pltpu.VMEM((bpw,), jnp.int32),
          pltpu.VMEM((bpw, d), jnp.float32),
          pltpu.VMEM((bpw, d), jnp.float32),
          pltpu.VMEM((bpw,), jnp.float32),
          pltpu.VMEM((bpw,), jnp.float32),
          pltpu.VMEM((L,), jnp.float32),
          pltpu.VMEM((bpw,), jnp.float32),
          pltpu.SemaphoreType.DMA,
      ],
  )
  return run(uidx, iidx, embedding_user, embedding_item,
             bias_user.astype(jnp.float32), bias_item.astype(jnp.float32),
             gm_arr)
